# baseline (speedup 1.0000x reference)
import jax
import jax.numpy as jnp
from jax import lax
from jax.experimental import pallas as pl
from jax.experimental.pallas import tpu as pltpu
from jax.experimental.pallas import tpu_sc as plsc


def _sc_assign(m):
    info = plsc.get_sparse_core_info()
    nc, ns, lanes = info.num_cores, info.num_subcores, info.num_lanes
    nw = nc * ns
    chunk = 640
    npad = nw * chunk
    groups = chunk // lanes

    mesh = plsc.VectorSubcoreMesh(core_axis_name="c", subcore_axis_name="s")

    def body(b_h, g_h, glab_h, obbox_h, olab_h,
             b_v, g_v, glab_v, o_v, olab_v, code_v):
        wid = lax.axis_index("s") * nc + lax.axis_index("c")
        base = wid * chunk
        pltpu.sync_copy(b_h.at[pl.ds(base * 4, chunk * 4)], b_v)
        pltpu.sync_copy(g_h, g_v)
        pltpu.sync_copy(glab_h, glab_v)

        iot4 = lax.iota(jnp.int32, lanes) * 4

        def group_body(g, _):
            off4 = g * lanes * 4
            bx1 = plsc.load_gather(b_v, [iot4 + off4])
            by1 = plsc.load_gather(b_v, [iot4 + (off4 + 1)])
            bx2 = plsc.load_gather(b_v, [iot4 + (off4 + 2)])
            by2 = plsc.load_gather(b_v, [iot4 + (off4 + 3)])
            area_b = (bx2 - bx1) * (by2 - by1)

            def gt_group(jg, code_min):
                goff4 = jg * lanes * 4
                gvx1 = plsc.load_gather(g_v, [iot4 + goff4])
                gvy1 = plsc.load_gather(g_v, [iot4 + (goff4 + 1)])
                gvx2 = plsc.load_gather(g_v, [iot4 + (goff4 + 2)])
                gvy2 = plsc.load_gather(g_v, [iot4 + (goff4 + 3)])
                gva = (gvx2 - gvx1) * (gvy2 - gvy1)
                goff = jg * lanes
                for k in range(lanes):
                    gx1, gy1 = gvx1[k], gvy1[k]
                    gx2, gy2 = gvx2[k], gvy2[k]
                    w = jnp.maximum(
                        jnp.minimum(bx2, gx2) - jnp.maximum(bx1, gx1), 0.0)
                    h = jnp.maximum(
                        jnp.minimum(by2, gy2) - jnp.maximum(by1, gy1), 0.0)
                    inter = w * h
                    union = jnp.maximum(area_b + gva[k] - inter, 1e-7)
                    code = jnp.where(inter >= 0.5 * union, goff + k,
                                     jnp.where(inter < 0.3 * union, 3 * m, m))
                    code_min = jnp.minimum(code_min, code)
                return code_min

            code_v[pl.ds(g * lanes, lanes)] = lax.fori_loop(
                0, m // lanes, gt_group,
                jnp.full((lanes,), 4 * m, jnp.int32))
            return 0

        lax.fori_loop(0, groups, group_body, 0)

        neg_one = jnp.float32(-1.0)
        for g in range(groups):
            off = g * lanes
            r = code_v[pl.ds(off, lanes)]
            pos_any = r < m
            neg_all = r >= 3 * m
            rc4 = jnp.where(pos_any, r, 0) * 4
            for c in range(4):
                sc = plsc.load_gather(g_v, [rc4 + c])
                plsc.store_scatter(o_v, [iot4 + (off * 4 + c)],
                                   jnp.where(pos_any, sc, neg_one))
            slab = plsc.load_gather(glab_v, [jnp.where(pos_any, r, 0)])
            olab_v[pl.ds(off, lanes)] = jnp.where(
                pos_any, slab,
                jnp.where(neg_all, jnp.int32(0), jnp.int32(-1)))

        pltpu.sync_copy(o_v, obbox_h.at[pl.ds(base * 4, chunk * 4)])
        pltpu.sync_copy(olab_v, olab_h.at[pl.ds(base, chunk)])

    f32 = jnp.float32
    i32 = jnp.int32
    out_type = [jax.ShapeDtypeStruct((npad * 4,), f32),
                jax.ShapeDtypeStruct((npad,), i32)]
    scratch = [
        pltpu.VMEM((chunk * 4,), f32),
        pltpu.VMEM((m * 4,), f32),
        pltpu.VMEM((m,), i32),
        pltpu.VMEM((chunk * 4,), f32),
        pltpu.VMEM((chunk,), i32),
        pltpu.VMEM((chunk,), i32),
    ]
    return pl.kernel(body, mesh=mesh, out_type=out_type,
                     scratch_types=scratch,
                     compiler_params=pltpu.CompilerParams(
                         needs_layout_passes=False)), npad


def kernel(bboxes, gt_bboxes, gt_labels):
    n = bboxes.shape[0]
    m = gt_bboxes.shape[0]
    sc_call, npad = _sc_assign(m)

    bflat = jnp.pad(bboxes.reshape(-1), (0, (npad - n) * 4))
    oflat, olab = sc_call(bflat, gt_bboxes.reshape(-1),
                          gt_labels.astype(jnp.int32))
    return olab[:n], oflat[:n * 4].reshape(n, 4)


# speedup vs baseline: 1.5107x; 1.5107x over previous
import jax
import jax.numpy as jnp
from jax import lax
from jax.experimental import pallas as pl
from jax.experimental.pallas import tpu as pltpu
from jax.experimental.pallas import tpu_sc as plsc


def _sc_assign(m):
    info = plsc.get_sparse_core_info()
    nc, ns, lanes = info.num_cores, info.num_subcores, info.num_lanes
    nw = nc * ns
    chunk = 640
    npad = nw * chunk
    groups = chunk // lanes

    mesh = plsc.VectorSubcoreMesh(core_axis_name="c", subcore_axis_name="s")

    def body(x1_h, y1_h, x2_h, y2_h, gx1_h, gy1_h, gx2_h, gy2_h, glab_h,
             ox1_h, oy1_h, ox2_h, oy2_h, olab_h,
             x1_v, y1_v, x2_v, y2_v, gx1_v, gy1_v, gx2_v, gy2_v, glab_v,
             ox1_v, oy1_v, ox2_v, oy2_v, olab_v, code_v):
        wid = lax.axis_index("s") * nc + lax.axis_index("c")
        base = wid * chunk
        pltpu.sync_copy(x1_h.at[pl.ds(base, chunk)], x1_v)
        pltpu.sync_copy(y1_h.at[pl.ds(base, chunk)], y1_v)
        pltpu.sync_copy(x2_h.at[pl.ds(base, chunk)], x2_v)
        pltpu.sync_copy(y2_h.at[pl.ds(base, chunk)], y2_v)
        pltpu.sync_copy(gx1_h, gx1_v)
        pltpu.sync_copy(gy1_h, gy1_v)
        pltpu.sync_copy(gx2_h, gx2_v)
        pltpu.sync_copy(gy2_h, gy2_v)
        pltpu.sync_copy(glab_h, glab_v)

        def group_body(g, _):
            off = g * lanes
            bx1 = x1_v[pl.ds(off, lanes)]
            by1 = y1_v[pl.ds(off, lanes)]
            bx2 = x2_v[pl.ds(off, lanes)]
            by2 = y2_v[pl.ds(off, lanes)]
            area_b = (bx2 - bx1) * (by2 - by1)

            def gt_group(jg, code_min):
                goff = jg * lanes
                gvx1 = gx1_v[pl.ds(goff, lanes)]
                gvy1 = gy1_v[pl.ds(goff, lanes)]
                gvx2 = gx2_v[pl.ds(goff, lanes)]
                gvy2 = gy2_v[pl.ds(goff, lanes)]
                gva = (gvx2 - gvx1) * (gvy2 - gvy1)
                for k in range(lanes):
                    gx1, gy1 = gvx1[k], gvy1[k]
                    gx2, gy2 = gvx2[k], gvy2[k]
                    w = jnp.maximum(
                        jnp.minimum(bx2, gx2) - jnp.maximum(bx1, gx1), 0.0)
                    h = jnp.maximum(
                        jnp.minimum(by2, gy2) - jnp.maximum(by1, gy1), 0.0)
                    inter = w * h
                    union = jnp.maximum(area_b + gva[k] - inter, 1e-7)
                    code = jnp.where(inter >= 0.5 * union, goff + k,
                                     jnp.where(inter < 0.3 * union, 3 * m, m))
                    code_min = jnp.minimum(code_min, code)
                return code_min

            code_v[pl.ds(off, lanes)] = lax.fori_loop(
                0, m // lanes, gt_group,
                jnp.full((lanes,), 4 * m, jnp.int32))
            return 0

        lax.fori_loop(0, groups, group_body, 0)

        neg_one = jnp.float32(-1.0)
        for g in range(groups):
            off = g * lanes
            r = code_v[pl.ds(off, lanes)]
            pos_any = r < m
            neg_all = r >= 3 * m
            rc = jnp.where(pos_any, r, 0)
            sx1 = plsc.load_gather(gx1_v, [rc])
            sy1 = plsc.load_gather(gy1_v, [rc])
            sx2 = plsc.load_gather(gx2_v, [rc])
            sy2 = plsc.load_gather(gy2_v, [rc])
            slab = plsc.load_gather(glab_v, [rc])
            ox1_v[pl.ds(off, lanes)] = jnp.where(pos_any, sx1, neg_one)
            oy1_v[pl.ds(off, lanes)] = jnp.where(pos_any, sy1, neg_one)
            ox2_v[pl.ds(off, lanes)] = jnp.where(pos_any, sx2, neg_one)
            oy2_v[pl.ds(off, lanes)] = jnp.where(pos_any, sy2, neg_one)
            olab_v[pl.ds(off, lanes)] = jnp.where(
                pos_any, slab,
                jnp.where(neg_all, jnp.int32(0), jnp.int32(-1)))

        pltpu.sync_copy(ox1_v, ox1_h.at[pl.ds(base, chunk)])
        pltpu.sync_copy(oy1_v, oy1_h.at[pl.ds(base, chunk)])
        pltpu.sync_copy(ox2_v, ox2_h.at[pl.ds(base, chunk)])
        pltpu.sync_copy(oy2_v, oy2_h.at[pl.ds(base, chunk)])
        pltpu.sync_copy(olab_v, olab_h.at[pl.ds(base, chunk)])

    f32 = jnp.float32
    i32 = jnp.int32
    out_type = [jax.ShapeDtypeStruct((npad,), f32)] * 4 + [
        jax.ShapeDtypeStruct((npad,), i32)]
    scratch = (
        [pltpu.VMEM((chunk,), f32)] * 4
        + [pltpu.VMEM((m,), f32)] * 4
        + [pltpu.VMEM((m,), i32)]
        + [pltpu.VMEM((chunk,), f32)] * 4
        + [pltpu.VMEM((chunk,), i32)] * 2
    )
    return pl.kernel(body, mesh=mesh, out_type=out_type,
                     scratch_types=scratch,
                     compiler_params=pltpu.CompilerParams(
                         needs_layout_passes=False)), npad


def kernel(bboxes, gt_bboxes, gt_labels):
    n = bboxes.shape[0]
    m = gt_bboxes.shape[0]
    sc_call, npad = _sc_assign(m)

    pad = npad - n
    x1 = jnp.pad(bboxes[:, 0], (0, pad))
    y1 = jnp.pad(bboxes[:, 1], (0, pad))
    x2 = jnp.pad(bboxes[:, 2], (0, pad))
    y2 = jnp.pad(bboxes[:, 3], (0, pad))

    ox1, oy1, ox2, oy2, olab = sc_call(
        x1, y1, x2, y2,
        gt_bboxes[:, 0], gt_bboxes[:, 1], gt_bboxes[:, 2], gt_bboxes[:, 3],
        gt_labels.astype(jnp.int32))

    assigned_bboxes = jnp.stack([ox1[:n], oy1[:n], ox2[:n], oy2[:n]], axis=1)
    return olab[:n], assigned_bboxes


# speedup vs baseline: 1.5178x; 1.0047x over previous
import jax
import jax.numpy as jnp
from jax import lax
from jax.experimental import pallas as pl
from jax.experimental.pallas import tpu as pltpu
from jax.experimental.pallas import tpu_sc as plsc


def _sc_assign(m):
    info = plsc.get_sparse_core_info()
    nc, ns, lanes = info.num_cores, info.num_subcores, info.num_lanes
    nw = nc * ns
    chunk = 640
    npad = nw * chunk
    groups = chunk // lanes

    mesh = plsc.VectorSubcoreMesh(core_axis_name="c", subcore_axis_name="s")

    def body(x1_h, y1_h, x2_h, y2_h, gx1_h, gy1_h, gx2_h, gy2_h, glab_h,
             ox1_h, oy1_h, ox2_h, oy2_h, olab_h,
             x1_v, y1_v, x2_v, y2_v, gx1_v, gy1_v, gx2_v, gy2_v, glab_v,
             ox1_v, oy1_v, ox2_v, oy2_v, olab_v, code_v):
        wid = lax.axis_index("s") * nc + lax.axis_index("c")
        base = wid * chunk
        pltpu.sync_copy(x1_h.at[pl.ds(base, chunk)], x1_v)
        pltpu.sync_copy(y1_h.at[pl.ds(base, chunk)], y1_v)
        pltpu.sync_copy(x2_h.at[pl.ds(base, chunk)], x2_v)
        pltpu.sync_copy(y2_h.at[pl.ds(base, chunk)], y2_v)
        pltpu.sync_copy(gx1_h, gx1_v)
        pltpu.sync_copy(gy1_h, gy1_v)
        pltpu.sync_copy(gx2_h, gx2_v)
        pltpu.sync_copy(gy2_h, gy2_v)
        pltpu.sync_copy(glab_h, glab_v)

        def group_body(g, _):
            off = g * lanes
            bx1 = x1_v[pl.ds(off, lanes)]
            by1 = y1_v[pl.ds(off, lanes)]
            bx2 = x2_v[pl.ds(off, lanes)]
            by2 = y2_v[pl.ds(off, lanes)]
            area_b = (bx2 - bx1) * (by2 - by1)

            code_min = jnp.full((lanes,), 4 * m, jnp.int32)
            for jg in range(m // lanes):
                goff = jg * lanes
                gvx1 = gx1_v[pl.ds(goff, lanes)]
                gvy1 = gy1_v[pl.ds(goff, lanes)]
                gvx2 = gx2_v[pl.ds(goff, lanes)]
                gvy2 = gy2_v[pl.ds(goff, lanes)]
                gva = (gvx2 - gvx1) * (gvy2 - gvy1)
                for k in range(lanes):
                    gx1, gy1 = gvx1[k], gvy1[k]
                    gx2, gy2 = gvx2[k], gvy2[k]
                    w = jnp.maximum(
                        jnp.minimum(bx2, gx2) - jnp.maximum(bx1, gx1), 0.0)
                    h = jnp.maximum(
                        jnp.minimum(by2, gy2) - jnp.maximum(by1, gy1), 0.0)
                    inter = w * h
                    union = jnp.maximum(area_b + gva[k] - inter, 1e-7)
                    code = jnp.where(inter >= 0.5 * union, goff + k,
                                     jnp.where(inter < 0.3 * union, 3 * m, m))
                    code_min = jnp.minimum(code_min, code)

            code_v[pl.ds(off, lanes)] = code_min
            return 0

        lax.fori_loop(0, groups, group_body, 0)

        neg_one = jnp.float32(-1.0)
        for g in range(groups):
            off = g * lanes
            r = code_v[pl.ds(off, lanes)]
            pos_any = r < m
            neg_all = r >= 3 * m
            rc = jnp.where(pos_any, r, 0)
            sx1 = plsc.load_gather(gx1_v, [rc])
            sy1 = plsc.load_gather(gy1_v, [rc])
            sx2 = plsc.load_gather(gx2_v, [rc])
            sy2 = plsc.load_gather(gy2_v, [rc])
            slab = plsc.load_gather(glab_v, [rc])
            ox1_v[pl.ds(off, lanes)] = jnp.where(pos_any, sx1, neg_one)
            oy1_v[pl.ds(off, lanes)] = jnp.where(pos_any, sy1, neg_one)
            ox2_v[pl.ds(off, lanes)] = jnp.where(pos_any, sx2, neg_one)
            oy2_v[pl.ds(off, lanes)] = jnp.where(pos_any, sy2, neg_one)
            olab_v[pl.ds(off, lanes)] = jnp.where(
                pos_any, slab,
                jnp.where(neg_all, jnp.int32(0), jnp.int32(-1)))

        pltpu.sync_copy(ox1_v, ox1_h.at[pl.ds(base, chunk)])
        pltpu.sync_copy(oy1_v, oy1_h.at[pl.ds(base, chunk)])
        pltpu.sync_copy(ox2_v, ox2_h.at[pl.ds(base, chunk)])
        pltpu.sync_copy(oy2_v, oy2_h.at[pl.ds(base, chunk)])
        pltpu.sync_copy(olab_v, olab_h.at[pl.ds(base, chunk)])

    f32 = jnp.float32
    i32 = jnp.int32
    out_type = [jax.ShapeDtypeStruct((npad,), f32)] * 4 + [
        jax.ShapeDtypeStruct((npad,), i32)]
    scratch = (
        [pltpu.VMEM((chunk,), f32)] * 4
        + [pltpu.VMEM((m,), f32)] * 4
        + [pltpu.VMEM((m,), i32)]
        + [pltpu.VMEM((chunk,), f32)] * 4
        + [pltpu.VMEM((chunk,), i32)] * 2
    )
    return pl.kernel(body, mesh=mesh, out_type=out_type,
                     scratch_types=scratch,
                     compiler_params=pltpu.CompilerParams(
                         needs_layout_passes=False)), npad


def kernel(bboxes, gt_bboxes, gt_labels):
    n = bboxes.shape[0]
    m = gt_bboxes.shape[0]
    sc_call, npad = _sc_assign(m)

    pad = npad - n
    x1 = jnp.pad(bboxes[:, 0], (0, pad))
    y1 = jnp.pad(bboxes[:, 1], (0, pad))
    x2 = jnp.pad(bboxes[:, 2], (0, pad))
    y2 = jnp.pad(bboxes[:, 3], (0, pad))

    ox1, oy1, ox2, oy2, olab = sc_call(
        x1, y1, x2, y2,
        gt_bboxes[:, 0], gt_bboxes[:, 1], gt_bboxes[:, 2], gt_bboxes[:, 3],
        gt_labels.astype(jnp.int32))

    assigned_bboxes = jnp.stack([ox1[:n], oy1[:n], ox2[:n], oy2[:n]], axis=1)
    return olab[:n], assigned_bboxes


# speedup vs baseline: 1.7401x; 1.1465x over previous
import functools

import jax
import jax.numpy as jnp
from jax import lax
from jax.experimental import pallas as pl
from jax.experimental.pallas import tpu as pltpu
from jax.experimental.pallas import tpu_sc as plsc


def _sc_assign(m, chunk):
    info = plsc.get_sparse_core_info()
    nc, ns, lanes = info.num_cores, info.num_subcores, info.num_lanes
    nw = nc * ns
    npad = nw * chunk
    groups = chunk // lanes

    mesh = plsc.VectorSubcoreMesh(core_axis_name="c", subcore_axis_name="s")

    def body(x1_h, y1_h, x2_h, y2_h, gx1_h, gy1_h, gx2_h, gy2_h, glab_h,
             ox1_h, oy1_h, ox2_h, oy2_h, olab_h,
             x1_v, y1_v, x2_v, y2_v, gx1_v, gy1_v, gx2_v, gy2_v, glab_v,
             ox1_v, oy1_v, ox2_v, oy2_v, olab_v, code_v):
        wid = lax.axis_index("s") * nc + lax.axis_index("c")
        base = wid * chunk
        pltpu.sync_copy(x1_h.at[pl.ds(base, chunk)], x1_v)
        pltpu.sync_copy(y1_h.at[pl.ds(base, chunk)], y1_v)
        pltpu.sync_copy(x2_h.at[pl.ds(base, chunk)], x2_v)
        pltpu.sync_copy(y2_h.at[pl.ds(base, chunk)], y2_v)
        pltpu.sync_copy(gx1_h, gx1_v)
        pltpu.sync_copy(gy1_h, gy1_v)
        pltpu.sync_copy(gx2_h, gx2_v)
        pltpu.sync_copy(gy2_h, gy2_v)
        pltpu.sync_copy(glab_h, glab_v)

        def group_body(g, _):
            off = g * lanes
            bx1 = x1_v[pl.ds(off, lanes)]
            by1 = y1_v[pl.ds(off, lanes)]
            bx2 = x2_v[pl.ds(off, lanes)]
            by2 = y2_v[pl.ds(off, lanes)]
            area_b = (bx2 - bx1) * (by2 - by1)

            code_min = jnp.full((lanes,), 4 * m, jnp.int32)
            for jg in range(m // lanes):
                goff = jg * lanes
                gvx1 = gx1_v[pl.ds(goff, lanes)]
                gvy1 = gy1_v[pl.ds(goff, lanes)]
                gvx2 = gx2_v[pl.ds(goff, lanes)]
                gvy2 = gy2_v[pl.ds(goff, lanes)]
                gva = (gvx2 - gvx1) * (gvy2 - gvy1)
                for k in range(lanes):
                    gx1, gy1 = gvx1[k], gvy1[k]
                    gx2, gy2 = gvx2[k], gvy2[k]
                    w = jnp.maximum(
                        jnp.minimum(bx2, gx2) - jnp.maximum(bx1, gx1), 0.0)
                    h = jnp.maximum(
                        jnp.minimum(by2, gy2) - jnp.maximum(by1, gy1), 0.0)
                    inter = w * h
                    union = jnp.maximum(area_b + gva[k] - inter, 1e-7)
                    code = jnp.where(inter >= 0.5 * union, goff + k,
                                     jnp.where(inter < 0.3 * union, 3 * m, m))
                    code_min = jnp.minimum(code_min, code)

            code_v[pl.ds(off, lanes)] = code_min
            return 0

        lax.fori_loop(0, groups, group_body, 0)

        neg_one = jnp.float32(-1.0)
        for g in range(groups):
            off = g * lanes
            r = code_v[pl.ds(off, lanes)]
            pos_any = r < m
            neg_all = r >= 3 * m
            rc = jnp.where(pos_any, r, 0)
            sx1 = plsc.load_gather(gx1_v, [rc])
            sy1 = plsc.load_gather(gy1_v, [rc])
            sx2 = plsc.load_gather(gx2_v, [rc])
            sy2 = plsc.load_gather(gy2_v, [rc])
            slab = plsc.load_gather(glab_v, [rc])
            ox1_v[pl.ds(off, lanes)] = jnp.where(pos_any, sx1, neg_one)
            oy1_v[pl.ds(off, lanes)] = jnp.where(pos_any, sy1, neg_one)
            ox2_v[pl.ds(off, lanes)] = jnp.where(pos_any, sx2, neg_one)
            oy2_v[pl.ds(off, lanes)] = jnp.where(pos_any, sy2, neg_one)
            olab_v[pl.ds(off, lanes)] = jnp.where(
                pos_any, slab,
                jnp.where(neg_all, jnp.int32(0), jnp.int32(-1)))

        pltpu.sync_copy(ox1_v, ox1_h.at[pl.ds(base, chunk)])
        pltpu.sync_copy(oy1_v, oy1_h.at[pl.ds(base, chunk)])
        pltpu.sync_copy(ox2_v, ox2_h.at[pl.ds(base, chunk)])
        pltpu.sync_copy(oy2_v, oy2_h.at[pl.ds(base, chunk)])
        pltpu.sync_copy(olab_v, olab_h.at[pl.ds(base, chunk)])

    f32 = jnp.float32
    i32 = jnp.int32
    out_type = [jax.ShapeDtypeStruct((npad,), f32)] * 4 + [
        jax.ShapeDtypeStruct((npad,), i32)]
    scratch = (
        [pltpu.VMEM((chunk,), f32)] * 4
        + [pltpu.VMEM((m,), f32)] * 4
        + [pltpu.VMEM((m,), i32)]
        + [pltpu.VMEM((chunk,), f32)] * 4
        + [pltpu.VMEM((chunk,), i32)] * 2
    )
    return pl.kernel(body, mesh=mesh, out_type=out_type,
                     scratch_types=scratch,
                     compiler_params=pltpu.CompilerParams(
                         needs_layout_passes=False)), npad


def _tc_block(b_ref, g_ref, lab_ref, bbox_ref, out_lab_ref, *, m: int):
    b = b_ref[...]
    g4 = g_ref[...]
    g = g4.T
    labf = lab_ref[...].astype(jnp.float32)

    bx1, by1, bx2, by2 = b[:, 0:1], b[:, 1:2], b[:, 2:3], b[:, 3:4]
    gx1, gy1, gx2, gy2 = g[0:1, :], g[1:2, :], g[2:3, :], g[3:4, :]

    w = jnp.maximum(jnp.minimum(bx2, gx2) - jnp.maximum(bx1, gx1), 0.0)
    h = jnp.maximum(jnp.minimum(by2, gy2) - jnp.maximum(by1, gy1), 0.0)
    inter = w * h
    area_b = (bx2 - bx1) * (by2 - by1)
    area_g = (gx2 - gx1) * (gy2 - gy1)
    union = jnp.maximum(area_b + area_g - inter, 1e-7)
    lane = jax.lax.broadcasted_iota(jnp.int32, inter.shape, 1)
    code = jnp.where(inter >= 0.5 * union, lane,
                     jnp.where(inter < 0.3 * union, 3 * m, m))
    r = jnp.min(code, axis=1, keepdims=True)
    pos_any = r < m
    neg_all = r >= 3 * m

    onehot = (lane == r).astype(jnp.float32)
    sel_bbox = jnp.dot(onehot, g4, preferred_element_type=jnp.float32)
    sel_lab = jnp.dot(onehot, labf, preferred_element_type=jnp.float32)

    neg_one = jnp.float32(-1.0)
    bbox_ref[...] = jnp.where(pos_any, sel_bbox, neg_one)
    labf_out = jnp.where(pos_any, jnp.round(sel_lab),
                         jnp.where(neg_all, 0.0, neg_one))
    out_lab_ref[...] = labf_out.astype(jnp.int32)


def _tc_assign(bb, gt_bboxes, glab2d, tile):
    n = bb.shape[0]
    m = gt_bboxes.shape[0]
    grid = (n + tile - 1) // tile
    return pl.pallas_call(
        functools.partial(_tc_block, m=m),
        grid=(grid,),
        in_specs=[
            pl.BlockSpec((tile, 4), lambda i: (i, 0)),
            pl.BlockSpec((m, 4), lambda i: (0, 0)),
            pl.BlockSpec((m, 1), lambda i: (0, 0)),
        ],
        out_specs=[
            pl.BlockSpec((tile, 4), lambda i: (i, 0)),
            pl.BlockSpec((tile, 1), lambda i: (i, 0)),
        ],
        out_shape=[
            jax.ShapeDtypeStruct((n, 4), jnp.float32),
            jax.ShapeDtypeStruct((n, 1), jnp.int32),
        ],
    )(bb, gt_bboxes, glab2d)


def kernel(bboxes, gt_bboxes, gt_labels):
    n = bboxes.shape[0]
    m = gt_bboxes.shape[0]
    chunk = 352
    sc_call, n_sc = _sc_assign(m, chunk)
    n_tc = n - n_sc

    labels_i32 = gt_labels.astype(jnp.int32)
    ox1, oy1, ox2, oy2, olab_sc = sc_call(
        bboxes[:n_sc, 0], bboxes[:n_sc, 1],
        bboxes[:n_sc, 2], bboxes[:n_sc, 3],
        gt_bboxes[:, 0], gt_bboxes[:, 1], gt_bboxes[:, 2], gt_bboxes[:, 3],
        labels_i32)

    bbox_tc, lab_tc = _tc_assign(bboxes[n_sc:], gt_bboxes,
                                 labels_i32.reshape(m, 1), tile=2184)

    sc_bbox = jnp.stack([ox1, oy1, ox2, oy2], axis=1)
    assigned_labels = jnp.concatenate([olab_sc, lab_tc.reshape(n_tc)])
    assigned_bboxes = jnp.concatenate([sc_bbox, bbox_tc], axis=0)
    return assigned_labels, assigned_bboxes


# speedup vs baseline: 1.7516x; 1.0066x over previous
import functools

import jax
import jax.numpy as jnp
from jax import lax
from jax.experimental import pallas as pl
from jax.experimental.pallas import tpu as pltpu
from jax.experimental.pallas import tpu_sc as plsc


def _sc_assign(m, chunk):
    info = plsc.get_sparse_core_info()
    nc, ns, lanes = info.num_cores, info.num_subcores, info.num_lanes
    nw = nc * ns
    npad = nw * chunk
    groups = chunk // lanes

    mesh = plsc.VectorSubcoreMesh(core_axis_name="c", subcore_axis_name="s")

    def body(x1_h, y1_h, x2_h, y2_h, gx1_h, gy1_h, gx2_h, gy2_h, glab_h,
             ox1_h, oy1_h, ox2_h, oy2_h, olab_h,
             x1_v, y1_v, x2_v, y2_v, gx1_v, gy1_v, gx2_v, gy2_v, glab_v,
             ox1_v, oy1_v, ox2_v, oy2_v, olab_v, code_v):
        wid = lax.axis_index("s") * nc + lax.axis_index("c")
        base = wid * chunk
        pltpu.sync_copy(x1_h.at[pl.ds(base, chunk)], x1_v)
        pltpu.sync_copy(y1_h.at[pl.ds(base, chunk)], y1_v)
        pltpu.sync_copy(x2_h.at[pl.ds(base, chunk)], x2_v)
        pltpu.sync_copy(y2_h.at[pl.ds(base, chunk)], y2_v)
        pltpu.sync_copy(gx1_h, gx1_v)
        pltpu.sync_copy(gy1_h, gy1_v)
        pltpu.sync_copy(gx2_h, gx2_v)
        pltpu.sync_copy(gy2_h, gy2_v)
        pltpu.sync_copy(glab_h, glab_v)

        def group_body(g, _):
            off = g * lanes
            bx1 = x1_v[pl.ds(off, lanes)]
            by1 = y1_v[pl.ds(off, lanes)]
            bx2 = x2_v[pl.ds(off, lanes)]
            by2 = y2_v[pl.ds(off, lanes)]
            area_b = (bx2 - bx1) * (by2 - by1)

            code_min = jnp.full((lanes,), 4 * m, jnp.int32)
            for jg in range(m // lanes):
                goff = jg * lanes
                gvx1 = gx1_v[pl.ds(goff, lanes)]
                gvy1 = gy1_v[pl.ds(goff, lanes)]
                gvx2 = gx2_v[pl.ds(goff, lanes)]
                gvy2 = gy2_v[pl.ds(goff, lanes)]
                gva = (gvx2 - gvx1) * (gvy2 - gvy1)
                for k in range(lanes):
                    gx1, gy1 = gvx1[k], gvy1[k]
                    gx2, gy2 = gvx2[k], gvy2[k]
                    w = jnp.maximum(
                        jnp.minimum(bx2, gx2) - jnp.maximum(bx1, gx1), 0.0)
                    h = jnp.maximum(
                        jnp.minimum(by2, gy2) - jnp.maximum(by1, gy1), 0.0)
                    inter = w * h
                    union = jnp.maximum(area_b + gva[k] - inter, 1e-7)
                    code = jnp.where(inter >= 0.5 * union, goff + k,
                                     jnp.where(inter < 0.3 * union, 3 * m, m))
                    code_min = jnp.minimum(code_min, code)

            code_v[pl.ds(off, lanes)] = code_min
            return 0

        lax.fori_loop(0, groups, group_body, 0)

        neg_one = jnp.float32(-1.0)
        for g in range(groups):
            off = g * lanes
            r = code_v[pl.ds(off, lanes)]
            pos_any = r < m
            neg_all = r >= 3 * m
            rc = jnp.where(pos_any, r, 0)
            sx1 = plsc.load_gather(gx1_v, [rc])
            sy1 = plsc.load_gather(gy1_v, [rc])
            sx2 = plsc.load_gather(gx2_v, [rc])
            sy2 = plsc.load_gather(gy2_v, [rc])
            slab = plsc.load_gather(glab_v, [rc])
            ox1_v[pl.ds(off, lanes)] = jnp.where(pos_any, sx1, neg_one)
            oy1_v[pl.ds(off, lanes)] = jnp.where(pos_any, sy1, neg_one)
            ox2_v[pl.ds(off, lanes)] = jnp.where(pos_any, sx2, neg_one)
            oy2_v[pl.ds(off, lanes)] = jnp.where(pos_any, sy2, neg_one)
            olab_v[pl.ds(off, lanes)] = jnp.where(
                pos_any, slab,
                jnp.where(neg_all, jnp.int32(0), jnp.int32(-1)))

        pltpu.sync_copy(ox1_v, ox1_h.at[pl.ds(base, chunk)])
        pltpu.sync_copy(oy1_v, oy1_h.at[pl.ds(base, chunk)])
        pltpu.sync_copy(ox2_v, ox2_h.at[pl.ds(base, chunk)])
        pltpu.sync_copy(oy2_v, oy2_h.at[pl.ds(base, chunk)])
        pltpu.sync_copy(olab_v, olab_h.at[pl.ds(base, chunk)])

    f32 = jnp.float32
    i32 = jnp.int32
    out_type = [jax.ShapeDtypeStruct((npad,), f32)] * 4 + [
        jax.ShapeDtypeStruct((npad,), i32)]
    scratch = (
        [pltpu.VMEM((chunk,), f32)] * 4
        + [pltpu.VMEM((m,), f32)] * 4
        + [pltpu.VMEM((m,), i32)]
        + [pltpu.VMEM((chunk,), f32)] * 4
        + [pltpu.VMEM((chunk,), i32)] * 2
    )
    return pl.kernel(body, mesh=mesh, out_type=out_type,
                     scratch_types=scratch,
                     compiler_params=pltpu.CompilerParams(
                         needs_layout_passes=False)), npad


def _tc_block(b_ref, g_ref, lab_ref, bbox_ref, out_lab_ref, *, m: int):
    b = b_ref[...]
    g4 = g_ref[...]
    g = g4.T
    labf = lab_ref[...].astype(jnp.float32)

    bx1, by1, bx2, by2 = b[:, 0:1], b[:, 1:2], b[:, 2:3], b[:, 3:4]
    gx1, gy1, gx2, gy2 = g[0:1, :], g[1:2, :], g[2:3, :], g[3:4, :]

    w = jnp.maximum(jnp.minimum(bx2, gx2) - jnp.maximum(bx1, gx1), 0.0)
    h = jnp.maximum(jnp.minimum(by2, gy2) - jnp.maximum(by1, gy1), 0.0)
    inter = w * h
    area_b = (bx2 - bx1) * (by2 - by1)
    area_g = (gx2 - gx1) * (gy2 - gy1)
    union = jnp.maximum(area_b + area_g - inter, 1e-7)
    lane = jax.lax.broadcasted_iota(jnp.int32, inter.shape, 1)
    code = jnp.where(inter >= 0.5 * union, lane,
                     jnp.where(inter < 0.3 * union, 3 * m, m))
    r = jnp.min(code, axis=1, keepdims=True)
    pos_any = r < m
    neg_all = r >= 3 * m

    onehot = (lane == r).astype(jnp.float32)
    sel_bbox = jnp.dot(onehot, g4, preferred_element_type=jnp.float32)
    sel_lab = jnp.dot(onehot, labf, preferred_element_type=jnp.float32)

    neg_one = jnp.float32(-1.0)
    bbox_ref[...] = jnp.where(pos_any, sel_bbox, neg_one)
    labf_out = jnp.where(pos_any, jnp.round(sel_lab),
                         jnp.where(neg_all, 0.0, neg_one))
    out_lab_ref[...] = labf_out.astype(jnp.int32)


def _tc_assign(bb, gt_bboxes, glab2d, tile):
    n = bb.shape[0]
    m = gt_bboxes.shape[0]
    grid = (n + tile - 1) // tile
    return pl.pallas_call(
        functools.partial(_tc_block, m=m),
        grid=(grid,),
        in_specs=[
            pl.BlockSpec((tile, 4), lambda i: (i, 0)),
            pl.BlockSpec((m, 4), lambda i: (0, 0)),
            pl.BlockSpec((m, 1), lambda i: (0, 0)),
        ],
        out_specs=[
            pl.BlockSpec((tile, 4), lambda i: (i, 0)),
            pl.BlockSpec((tile, 1), lambda i: (i, 0)),
        ],
        out_shape=[
            jax.ShapeDtypeStruct((n, 4), jnp.float32),
            jax.ShapeDtypeStruct((n, 1), jnp.int32),
        ],
    )(bb, gt_bboxes, glab2d)


def kernel(bboxes, gt_bboxes, gt_labels):
    n = bboxes.shape[0]
    m = gt_bboxes.shape[0]
    chunk = 480
    sc_call, n_sc = _sc_assign(m, chunk)
    n_tc = n - n_sc

    labels_i32 = gt_labels.astype(jnp.int32)
    ox1, oy1, ox2, oy2, olab_sc = sc_call(
        bboxes[:n_sc, 0], bboxes[:n_sc, 1],
        bboxes[:n_sc, 2], bboxes[:n_sc, 3],
        gt_bboxes[:, 0], gt_bboxes[:, 1], gt_bboxes[:, 2], gt_bboxes[:, 3],
        labels_i32)

    bbox_tc, lab_tc = _tc_assign(bboxes[n_sc:], gt_bboxes,
                                 labels_i32.reshape(m, 1), tile=2320)

    sc_bbox = jnp.stack([ox1, oy1, ox2, oy2], axis=1)
    assigned_labels = jnp.concatenate([olab_sc, lab_tc.reshape(n_tc)])
    assigned_bboxes = jnp.concatenate([sc_bbox, bbox_tc], axis=0)
    return assigned_labels, assigned_bboxes
